# baseline (speedup 1.0000x reference)
import jax
import jax.numpy as jnp
from jax import lax
from jax.experimental import pallas as pl
from jax.experimental.pallas import tpu as pltpu
from jax.experimental.pallas import tpu_sc as plsc

N_NODES = 10000
N_EDGES = 160000
D_IN = 256
D_HID = 256
N_CLASSES = 16
N_GRAPHS = 64

NC = 2
NT = 16
CH = 128
N_PAD = 10240
E_PAD = 163840
MSG_CHUNKS = E_PAD // NT // CH
HCH = MSG_CHUNKS // 2
DEG_CHUNKS = E_PAD // (NC * NT) // CH
ROWS_PER_TILE = N_PAD // NT
RB = 512
N_BLOCKS = N_PAD // RB

_mesh = plsc.VectorSubcoreMesh(core_axis_name="c", subcore_axis_name="s")


def _deg_body(dst_hbm, out_hbm, dst_v, buf, acc):
    c = lax.axis_index("c")
    s = lax.axis_index("s")
    w = c * NT + s

    def zfill(i, carry):
        for k in range(CH // 16):
            buf[i, pl.ds(k * 16, 16)] = jnp.zeros((16,), jnp.float32)
        return carry

    lax.fori_loop(0, CH, zfill, 0)
    for k in range(ROWS_PER_TILE // CH):
        pltpu.sync_copy(buf, acc.at[pl.ds(s * ROWS_PER_TILE + k * CH, CH)])

    def ofill(i, carry):
        for k in range(CH // 16):
            buf[i, pl.ds(k * 16, 16)] = jnp.ones((16,), jnp.float32)
        return carry

    lax.fori_loop(0, CH, ofill, 0)
    pltpu.sync_copy(dst_hbm.at[w], dst_v)
    plsc.subcore_barrier()

    def scat(j, carry):
        pltpu.sync_copy(buf, acc.at[dst_v.at[j]], add=True)
        return carry

    lax.fori_loop(0, DEG_CHUNKS, scat, 0)
    plsc.subcore_barrier()
    for k in range(ROWS_PER_TILE // CH):
        r = s * ROWS_PER_TILE + k * CH
        pltpu.sync_copy(acc.at[pl.ds(r, CH)], out_hbm.at[c, pl.ds(r, CH)])


_deg_call = pl.kernel(
    _deg_body,
    out_type=jax.ShapeDtypeStruct((NC, N_PAD, CH), jnp.float32),
    mesh=_mesh,
    scratch_types=[
        pltpu.VMEM((DEG_CHUNKS, CH), jnp.int32),
        pltpu.VMEM((CH, CH), jnp.float32),
        pltpu.VMEM_SHARED((N_PAD, CH), jnp.float32),
    ],
)


def _msg_body(z_hbm, src_hbm, dst_hbm, out_hbm,
              src_v, dst_v, buf0, buf1, sem0, sem1, ssem0, ssem1, acc):
    c = lax.axis_index("c")
    s = lax.axis_index("s")
    base_off = c * N_PAD

    def zfill(i, carry):
        for k in range(CH // 16):
            buf0[i, pl.ds(k * 16, 16)] = jnp.zeros((16,), jnp.float32)
        return carry

    lax.fori_loop(0, CH, zfill, 0)
    for k in range(ROWS_PER_TILE // CH):
        pltpu.sync_copy(buf0, acc.at[pl.ds(s * ROWS_PER_TILE + k * CH, CH)])
    plsc.subcore_barrier()

    for h in range(MSG_CHUNKS // HCH):
        pltpu.sync_copy(src_hbm.at[s, pl.ds(h * HCH, HCH)], src_v)
        pltpu.sync_copy(dst_hbm.at[s, pl.ds(h * HCH, HCH)], dst_v)

        def off(i, carry):
            for k in range(CH // 16):
                sl = pl.ds(k * 16, 16)
                src_v[i, sl] = src_v[i, sl] + base_off
            return carry

        lax.fori_loop(0, HCH, off, 0)

        pltpu.async_copy(z_hbm.at[src_v.at[0]], buf0, sem0)
        pltpu.async_copy(z_hbm.at[src_v.at[1]], buf1, sem1)

        def body(g, carry):
            j0 = 2 * g
            j1 = j0 + 1
            pltpu.make_async_copy(z_hbm.at[src_v.at[j0]], buf0, sem0).wait()
            pltpu.async_copy(buf0, acc.at[dst_v.at[j0]], ssem0, add=True)
            pltpu.make_async_copy(z_hbm.at[src_v.at[j1]], buf1, sem1).wait()
            pltpu.async_copy(buf1, acc.at[dst_v.at[j1]], ssem1, add=True)

            @pl.when(j0 + 2 < HCH)
            def _():
                pltpu.make_async_copy(
                    buf0, acc.at[dst_v.at[j0]], ssem0).wait()
                pltpu.async_copy(z_hbm.at[src_v.at[j0 + 2]], buf0, sem0)

            @pl.when(j1 + 2 < HCH)
            def _():
                pltpu.make_async_copy(
                    buf1, acc.at[dst_v.at[j1]], ssem1).wait()
                pltpu.async_copy(z_hbm.at[src_v.at[j1 + 2]], buf1, sem1)

            return carry

        lax.fori_loop(0, HCH // 2, body, 0)
        pltpu.make_async_copy(buf0, acc.at[dst_v.at[HCH - 2]], ssem0).wait()
        pltpu.make_async_copy(buf1, acc.at[dst_v.at[HCH - 1]], ssem1).wait()
    plsc.subcore_barrier()
    for k in range(ROWS_PER_TILE // CH):
        r = s * ROWS_PER_TILE + k * CH
        pltpu.sync_copy(acc.at[pl.ds(r, CH)],
                        out_hbm.at[pl.ds(base_off + r, CH)])


_msg_call = pl.kernel(
    _msg_body,
    out_type=jax.ShapeDtypeStruct((NC * N_PAD, 128), jnp.float32),
    mesh=_mesh,
    scratch_types=[
        pltpu.VMEM((HCH, CH), jnp.int32),
        pltpu.VMEM((HCH, CH), jnp.int32),
        pltpu.VMEM((CH, 128), jnp.float32),
        pltpu.VMEM((CH, 128), jnp.float32),
        pltpu.SemaphoreType.DMA,
        pltpu.SemaphoreType.DMA,
        pltpu.SemaphoreType.DMA,
        pltpu.SemaphoreType.DMA,
        pltpu.VMEM_SHARED((N_PAD, 128), jnp.float32),
    ],
)


def _dinv(dp_ref):
    deg = dp_ref[0, :, 0:1] + dp_ref[1, :, 0:1] + 1.0
    return lax.rsqrt(deg)


def _zw_body(x_ref, w_ref, dp_ref, o_ref):
    z = jnp.dot(x_ref[...], w_ref[...],
                preferred_element_type=jnp.float32) * _dinv(dp_ref)
    o_ref[0] = z[:, :128]
    o_ref[1] = z[:, 128:]


_zw_call = pl.pallas_call(
    _zw_body,
    grid=(N_BLOCKS,),
    in_specs=[
        pl.BlockSpec((RB, D_IN), lambda i: (i, 0)),
        pl.BlockSpec((D_IN, D_HID), lambda i: (0, 0)),
        pl.BlockSpec((NC, RB, CH), lambda i: (0, i, 0)),
    ],
    out_specs=pl.BlockSpec((NC, RB, 128), lambda i: (0, i, 0)),
    out_shape=jax.ShapeDtypeStruct((NC, N_PAD, 128), jnp.float32),
)


def _h_body(s_ref, z_ref, dp_ref, b_ref, w_ref, o_ref):
    dinv = _dinv(dp_ref)
    t = jnp.concatenate([s_ref[0] + z_ref[0], s_ref[1] + z_ref[1]], axis=1)
    h = jnp.maximum(t * dinv + b_ref[...], 0.0)
    z2 = jnp.dot(h, w_ref[...], preferred_element_type=jnp.float32) * dinv
    o_ref[0] = z2[:, :128]
    o_ref[1] = z2[:, 128:]


_h_call = pl.pallas_call(
    _h_body,
    grid=(N_BLOCKS,),
    in_specs=[
        pl.BlockSpec((NC, RB, 128), lambda i: (0, i, 0)),
        pl.BlockSpec((NC, RB, 128), lambda i: (0, i, 0)),
        pl.BlockSpec((NC, RB, CH), lambda i: (0, i, 0)),
        pl.BlockSpec((1, D_HID), lambda i: (0, 0)),
        pl.BlockSpec((D_HID, D_HID), lambda i: (0, 0)),
    ],
    out_specs=pl.BlockSpec((NC, RB, 128), lambda i: (0, i, 0)),
    out_shape=jax.ShapeDtypeStruct((NC, N_PAD, 128), jnp.float32),
)


def _fin_body(s_ref, z_ref, dp_ref, b_ref, batch_ref, wfc_ref, bfc_ref,
              o_ref, acc_ref):
    i = pl.program_id(0)
    dinv = _dinv(dp_ref)
    t = jnp.concatenate([s_ref[0] + z_ref[0], s_ref[1] + z_ref[1]], axis=1)
    h = jnp.maximum(t * dinv + b_ref[...], 0.0)
    hb = jnp.concatenate([h, jnp.ones((RB, 128), jnp.float32)], axis=1)
    gi = lax.broadcasted_iota(jnp.int32, (N_GRAPHS, RB), 0)
    onehot = (batch_ref[...] == gi).astype(jnp.float32)

    @pl.when(i == 0)
    def _():
        acc_ref[...] = jnp.zeros_like(acc_ref)

    acc_ref[...] += jnp.dot(onehot, hb, preferred_element_type=jnp.float32)

    @pl.when(i == N_BLOCKS - 1)
    def _():
        cnt = jnp.maximum(acc_ref[:, D_HID:D_HID + 1], 1.0)
        pooled = acc_ref[:, :D_HID] / cnt
        o_ref[...] = jnp.dot(pooled, wfc_ref[...],
                             preferred_element_type=jnp.float32) + bfc_ref[...]


_fin_call = pl.pallas_call(
    _fin_body,
    grid=(N_BLOCKS,),
    in_specs=[
        pl.BlockSpec((NC, RB, 128), lambda i: (0, i, 0)),
        pl.BlockSpec((NC, RB, 128), lambda i: (0, i, 0)),
        pl.BlockSpec((NC, RB, CH), lambda i: (0, i, 0)),
        pl.BlockSpec((1, D_HID), lambda i: (0, 0)),
        pl.BlockSpec((1, RB), lambda i: (0, i)),
        pl.BlockSpec((D_HID, N_CLASSES), lambda i: (0, 0)),
        pl.BlockSpec((1, N_CLASSES), lambda i: (0, 0)),
    ],
    out_specs=pl.BlockSpec((N_GRAPHS, N_CLASSES), lambda i: (0, 0)),
    out_shape=jax.ShapeDtypeStruct((N_GRAPHS, N_CLASSES), jnp.float32),
    scratch_shapes=[pltpu.VMEM((N_GRAPHS, D_HID + 128), jnp.float32)],
)


def kernel(x, edge_index, batch, W1, b1, W2, b2, Wfc, bfc):
    src = edge_index[0].astype(jnp.int32)
    dst = edge_index[1].astype(jnp.int32)
    pad_e = E_PAD - N_EDGES
    src_p = jnp.concatenate([src, jnp.zeros((pad_e,), jnp.int32)])
    dst_p = jnp.concatenate([dst, jnp.full((pad_e,), N_NODES, jnp.int32)])
    src_tbl = src_p.reshape(NT, MSG_CHUNKS, CH)
    dst_tbl = dst_p.reshape(NT, MSG_CHUNKS, CH)
    dst_deg = dst_p.reshape(NC * NT, DEG_CHUNKS, CH)

    x_p = jnp.pad(x, ((0, N_PAD - N_NODES), (0, 0)))
    batch_p = jnp.pad(batch.astype(jnp.int32), (0, N_PAD - N_NODES),
                      constant_values=N_GRAPHS).reshape(1, N_PAD)
    b1r = b1.reshape(1, D_HID)
    b2r = b2.reshape(1, D_HID)
    bfcr = bfc.reshape(1, N_CLASSES)

    dp = _deg_call(dst_deg)
    z1 = _zw_call(x_p, W1, dp)
    S1 = _msg_call(z1.reshape(NC * N_PAD, 128), src_tbl, dst_tbl)
    z2 = _h_call(S1.reshape(NC, N_PAD, 128), z1, dp, b1r, W2)
    S2 = _msg_call(z2.reshape(NC * N_PAD, 128), src_tbl, dst_tbl)
    out = _fin_call(S2.reshape(NC, N_PAD, 128), z2, dp, b2r, batch_p,
                    Wfc, bfcr)
    return out


# speedup vs baseline: 1.0713x; 1.0713x over previous
import jax
import jax.numpy as jnp
from jax import lax
from jax.experimental import pallas as pl
from jax.experimental.pallas import tpu as pltpu
from jax.experimental.pallas import tpu_sc as plsc

N_NODES = 10000
N_EDGES = 160000
D_IN = 256
D_HID = 256
N_CLASSES = 16
N_GRAPHS = 64

NC = 2
NT = 16
CH = 128
N_PAD = 10240
E_PAD = 163840
MSG_CHUNKS = E_PAD // NT // CH
HCH = MSG_CHUNKS // 2
DEG_CHUNKS = E_PAD // (NC * NT) // CH
ROWS_PER_TILE = N_PAD // NT
RB = 512
N_BLOCKS = N_PAD // RB

_mesh = plsc.VectorSubcoreMesh(core_axis_name="c", subcore_axis_name="s")


def _deg_body(dst_hbm, out_hbm, dst_v, buf, sem, acc):
    c = lax.axis_index("c")
    s = lax.axis_index("s")
    w = c * NT + s

    def zfill(i, carry):
        for k in range(CH // 16):
            buf[i, pl.ds(k * 16, 16)] = jnp.zeros((16,), jnp.float32)
        return carry

    lax.fori_loop(0, CH, zfill, 0)
    for k in range(ROWS_PER_TILE // CH):
        pltpu.sync_copy(buf, acc.at[pl.ds(s * ROWS_PER_TILE + k * CH, CH)])

    def ofill(i, carry):
        for k in range(CH // 16):
            buf[i, pl.ds(k * 16, 16)] = jnp.ones((16,), jnp.float32)
        return carry

    lax.fori_loop(0, CH, ofill, 0)
    pltpu.sync_copy(dst_hbm.at[w], dst_v)
    plsc.subcore_barrier()

    def scat(j, carry):
        pltpu.async_copy(buf, acc.at[dst_v.at[j]], sem, add=True)
        return carry

    lax.fori_loop(0, DEG_CHUNKS, scat, 0)

    def drain(j, carry):
        pltpu.make_async_copy(buf, acc.at[dst_v.at[0]], sem).wait()
        return carry

    lax.fori_loop(0, DEG_CHUNKS, drain, 0)
    plsc.subcore_barrier()
    for k in range(ROWS_PER_TILE // CH):
        r = s * ROWS_PER_TILE + k * CH
        pltpu.sync_copy(acc.at[pl.ds(r, CH)], out_hbm.at[c, pl.ds(r, CH)])


_deg_call = pl.kernel(
    _deg_body,
    out_type=jax.ShapeDtypeStruct((NC, N_PAD, CH), jnp.float32),
    mesh=_mesh,
    scratch_types=[
        pltpu.VMEM((DEG_CHUNKS, CH), jnp.int32),
        pltpu.VMEM((CH, CH), jnp.float32),
        pltpu.SemaphoreType.DMA,
        pltpu.VMEM_SHARED((N_PAD, CH), jnp.float32),
    ],
)


def _msg_body(z_hbm, src_hbm, dst_hbm, out_hbm,
              src_v, dst_v, buf0, buf1, sem0, sem1, acc):
    c = lax.axis_index("c")
    s = lax.axis_index("s")
    base_off = c * N_PAD

    def zfill(i, carry):
        for k in range(CH // 16):
            buf0[i, pl.ds(k * 16, 16)] = jnp.zeros((16,), jnp.float32)
        return carry

    lax.fori_loop(0, CH, zfill, 0)
    for k in range(ROWS_PER_TILE // CH):
        pltpu.sync_copy(buf0, acc.at[pl.ds(s * ROWS_PER_TILE + k * CH, CH)])
    plsc.subcore_barrier()

    for h in range(MSG_CHUNKS // HCH):
        pltpu.sync_copy(src_hbm.at[s, pl.ds(h * HCH, HCH)], src_v)
        pltpu.sync_copy(dst_hbm.at[s, pl.ds(h * HCH, HCH)], dst_v)

        def off(i, carry):
            for k in range(CH // 16):
                sl = pl.ds(k * 16, 16)
                src_v[i, sl] = src_v[i, sl] + base_off
            return carry

        lax.fori_loop(0, HCH, off, 0)

        pltpu.async_copy(z_hbm.at[src_v.at[0]], buf0, sem0)
        pltpu.async_copy(z_hbm.at[src_v.at[1]], buf1, sem1)

        def body(g, carry):
            j0 = 2 * g
            pltpu.make_async_copy(z_hbm.at[src_v.at[j0]], buf0, sem0).wait()
            pltpu.sync_copy(buf0, acc.at[dst_v.at[j0]], add=True)

            @pl.when(j0 + 2 < HCH)
            def _():
                pltpu.async_copy(z_hbm.at[src_v.at[j0 + 2]], buf0, sem0)

            j1 = j0 + 1
            pltpu.make_async_copy(z_hbm.at[src_v.at[j1]], buf1, sem1).wait()
            pltpu.sync_copy(buf1, acc.at[dst_v.at[j1]], add=True)

            @pl.when(j1 + 2 < HCH)
            def _():
                pltpu.async_copy(z_hbm.at[src_v.at[j1 + 2]], buf1, sem1)

            return carry

        lax.fori_loop(0, HCH // 2, body, 0)
    plsc.subcore_barrier()
    for k in range(ROWS_PER_TILE // CH):
        r = s * ROWS_PER_TILE + k * CH
        pltpu.sync_copy(acc.at[pl.ds(r, CH)],
                        out_hbm.at[pl.ds(base_off + r, CH)])


_msg_call = pl.kernel(
    _msg_body,
    out_type=jax.ShapeDtypeStruct((NC * N_PAD, 128), jnp.float32),
    mesh=_mesh,
    scratch_types=[
        pltpu.VMEM((HCH, CH), jnp.int32),
        pltpu.VMEM((HCH, CH), jnp.int32),
        pltpu.VMEM((CH, 128), jnp.float32),
        pltpu.VMEM((CH, 128), jnp.float32),
        pltpu.SemaphoreType.DMA,
        pltpu.SemaphoreType.DMA,
        pltpu.VMEM_SHARED((N_PAD, 128), jnp.float32),
    ],
)


def _dinv(dp_ref):
    deg = dp_ref[0, :, 0:1] + dp_ref[1, :, 0:1] + 1.0
    return lax.rsqrt(deg)


def _zw_body(x_ref, w_ref, dp_ref, o_ref):
    z = jnp.dot(x_ref[...], w_ref[...],
                preferred_element_type=jnp.float32) * _dinv(dp_ref)
    o_ref[0] = z[:, :128]
    o_ref[1] = z[:, 128:]


_zw_call = pl.pallas_call(
    _zw_body,
    grid=(N_BLOCKS,),
    in_specs=[
        pl.BlockSpec((RB, D_IN), lambda i: (i, 0)),
        pl.BlockSpec((D_IN, D_HID), lambda i: (0, 0)),
        pl.BlockSpec((NC, RB, CH), lambda i: (0, i, 0)),
    ],
    out_specs=pl.BlockSpec((NC, RB, 128), lambda i: (0, i, 0)),
    out_shape=jax.ShapeDtypeStruct((NC, N_PAD, 128), jnp.float32),
)


def _h_body(s_ref, z_ref, dp_ref, b_ref, w_ref, o_ref):
    dinv = _dinv(dp_ref)
    t = jnp.concatenate([s_ref[0] + z_ref[0], s_ref[1] + z_ref[1]], axis=1)
    h = jnp.maximum(t * dinv + b_ref[...], 0.0)
    z2 = jnp.dot(h, w_ref[...], preferred_element_type=jnp.float32) * dinv
    o_ref[0] = z2[:, :128]
    o_ref[1] = z2[:, 128:]


_h_call = pl.pallas_call(
    _h_body,
    grid=(N_BLOCKS,),
    in_specs=[
        pl.BlockSpec((NC, RB, 128), lambda i: (0, i, 0)),
        pl.BlockSpec((NC, RB, 128), lambda i: (0, i, 0)),
        pl.BlockSpec((NC, RB, CH), lambda i: (0, i, 0)),
        pl.BlockSpec((1, D_HID), lambda i: (0, 0)),
        pl.BlockSpec((D_HID, D_HID), lambda i: (0, 0)),
    ],
    out_specs=pl.BlockSpec((NC, RB, 128), lambda i: (0, i, 0)),
    out_shape=jax.ShapeDtypeStruct((NC, N_PAD, 128), jnp.float32),
)


def _fin_body(s_ref, z_ref, dp_ref, b_ref, batch_ref, wfc_ref, bfc_ref,
              o_ref, acc_ref):
    i = pl.program_id(0)
    dinv = _dinv(dp_ref)
    t = jnp.concatenate([s_ref[0] + z_ref[0], s_ref[1] + z_ref[1]], axis=1)
    h = jnp.maximum(t * dinv + b_ref[...], 0.0)
    hb = jnp.concatenate([h, jnp.ones((RB, 128), jnp.float32)], axis=1)
    gi = lax.broadcasted_iota(jnp.int32, (N_GRAPHS, RB), 0)
    onehot = (batch_ref[...] == gi).astype(jnp.float32)

    @pl.when(i == 0)
    def _():
        acc_ref[...] = jnp.zeros_like(acc_ref)

    acc_ref[...] += jnp.dot(onehot, hb, preferred_element_type=jnp.float32)

    @pl.when(i == N_BLOCKS - 1)
    def _():
        cnt = jnp.maximum(acc_ref[:, D_HID:D_HID + 1], 1.0)
        pooled = acc_ref[:, :D_HID] / cnt
        o_ref[...] = jnp.dot(pooled, wfc_ref[...],
                             preferred_element_type=jnp.float32) + bfc_ref[...]


_fin_call = pl.pallas_call(
    _fin_body,
    grid=(N_BLOCKS,),
    in_specs=[
        pl.BlockSpec((NC, RB, 128), lambda i: (0, i, 0)),
        pl.BlockSpec((NC, RB, 128), lambda i: (0, i, 0)),
        pl.BlockSpec((NC, RB, CH), lambda i: (0, i, 0)),
        pl.BlockSpec((1, D_HID), lambda i: (0, 0)),
        pl.BlockSpec((1, RB), lambda i: (0, i)),
        pl.BlockSpec((D_HID, N_CLASSES), lambda i: (0, 0)),
        pl.BlockSpec((1, N_CLASSES), lambda i: (0, 0)),
    ],
    out_specs=pl.BlockSpec((N_GRAPHS, N_CLASSES), lambda i: (0, 0)),
    out_shape=jax.ShapeDtypeStruct((N_GRAPHS, N_CLASSES), jnp.float32),
    scratch_shapes=[pltpu.VMEM((N_GRAPHS, D_HID + 128), jnp.float32)],
)


def kernel(x, edge_index, batch, W1, b1, W2, b2, Wfc, bfc):
    src = edge_index[0].astype(jnp.int32)
    dst = edge_index[1].astype(jnp.int32)
    pad_e = E_PAD - N_EDGES
    src_p = jnp.concatenate([src, jnp.zeros((pad_e,), jnp.int32)])
    dst_p = jnp.concatenate([dst, jnp.full((pad_e,), N_NODES, jnp.int32)])
    src_tbl = src_p.reshape(NT, MSG_CHUNKS, CH)
    dst_tbl = dst_p.reshape(NT, MSG_CHUNKS, CH)
    dst_deg = dst_p.reshape(NC * NT, DEG_CHUNKS, CH)

    x_p = jnp.pad(x, ((0, N_PAD - N_NODES), (0, 0)))
    batch_p = jnp.pad(batch.astype(jnp.int32), (0, N_PAD - N_NODES),
                      constant_values=N_GRAPHS).reshape(1, N_PAD)
    b1r = b1.reshape(1, D_HID)
    b2r = b2.reshape(1, D_HID)
    bfcr = bfc.reshape(1, N_CLASSES)

    dp = _deg_call(dst_deg)
    z1 = _zw_call(x_p, W1, dp)
    S1 = _msg_call(z1.reshape(NC * N_PAD, 128), src_tbl, dst_tbl)
    z2 = _h_call(S1.reshape(NC, N_PAD, 128), z1, dp, b1r, W2)
    S2 = _msg_call(z2.reshape(NC * N_PAD, 128), src_tbl, dst_tbl)
    out = _fin_call(S2.reshape(NC, N_PAD, 128), z2, dp, b2r, batch_p,
                    Wfc, bfcr)
    return out
